# baseline (speedup 1.0000x reference)
import functools

import jax
import jax.numpy as jnp
from jax import lax
from jax.experimental import pallas as pl
from jax.experimental.pallas import tpu as pltpu
from jax.experimental.pallas import tpu_sc as plsc

NUM_CORES = 2
NUM_SUBCORES = 16
NW = NUM_CORES * NUM_SUBCORES
LANES = 16
B = 16384
D = 64
TW = 2 * D
V = 1000000
BPW = B // NW
CHUNK = 128
NCHUNK = BPW // CHUNK
PASS_CHUNKS = 2
ROWS_PER_PASS = PASS_CHUNKS * CHUNK
NPASS = NCHUNK // PASS_CHUNKS
BLK_PER_PASS = ROWS_PER_PASS // LANES

TRW = 512
TRG = -(-V // (2 * TRW))
PACKED_ROWS = TRG * TRW


def _tr_body(in_lo_ref, in_hi_ref, out_ref):
    out_ref[:, 0:D] = in_lo_ref[...].T
    out_ref[:, D:TW] = in_hi_ref[...].T


def _transpose_table(tab_t):
    return pl.pallas_call(
        _tr_body,
        grid=(TRG,),
        in_specs=[
            pl.BlockSpec((D, TRW), lambda c: (0, 2 * c)),
            pl.BlockSpec((D, TRW), lambda c: (0, 2 * c + 1)),
        ],
        out_specs=pl.BlockSpec((TRW, TW), lambda c: (c, 0)),
        out_shape=jax.ShapeDtypeStruct((PACKED_ROWS, TW), jnp.float32),
    )(tab_t, tab_t)


def _packed_row(i):
    return lax.shift_left(lax.shift_right_logical(i, 10), 9) + (i & 511)


def _col_base(i):
    return lax.shift_left(lax.shift_right_logical(i, 9) & 1, 6)


def _mf_body(users_hbm, items_hbm, tab_u_hbm, tab_i_hbm, out_hbm,
             idx_u, idx_i, row_u, row_i, rows_u, rows_i, out_v, sem):
    wid = lax.axis_index("c") * NUM_SUBCORES + lax.axis_index("s")
    base = wid * BPW

    pltpu.sync_copy(users_hbm.at[wid], idx_u)
    pltpu.sync_copy(items_hbm.at[wid], idx_i)
    for j in range(NCHUNK):
        for k in range(CHUNK // LANES):
            s = pl.ds(k * LANES, LANES)
            row_u[j, s] = _packed_row(idx_u[j, s])
            row_i[j, s] = _packed_row(idx_i[j, s])

    for p in range(NPASS):
        copies = []
        for j in range(PASS_CHUNKS):
            c = p * PASS_CHUNKS + j
            dst = pl.ds(j * CHUNK, CHUNK)
            copies.append(pltpu.async_copy(
                tab_u_hbm.at[row_u.at[c]], rows_u.at[dst], sem))
            copies.append(pltpu.async_copy(
                tab_i_hbm.at[row_i.at[c]], rows_i.at[dst], sem))
        for cp in copies:
            cp.wait()

        for blk in range(BLK_PER_PASS):
            g = p * ROWS_PER_PASS + blk * LANES
            row_ids = blk * LANES + lax.iota(jnp.int32, LANES)
            cb_u = _col_base(idx_u[g // CHUNK, pl.ds(g % CHUNK, LANES)])
            cb_i = _col_base(idx_i[g // CHUNK, pl.ds(g % CHUNK, LANES)])
            acc = jnp.zeros((LANES,), jnp.float32)
            for d in range(D):
                u = plsc.load_gather(rows_u, [row_ids, cb_u + d])
                v = plsc.load_gather(rows_i, [row_ids, cb_i + d])
                acc = acc + u * v
            out_v[pl.ds(g, LANES)] = 1.0 / (1.0 + jnp.exp(-acc))

    pltpu.sync_copy(out_v, out_hbm.at[pl.ds(base, BPW)])


@jax.jit
def _mf_call(users_r, items_r, tab_u, tab_i):
    mesh = plsc.VectorSubcoreMesh(core_axis_name="c", subcore_axis_name="s")
    run = functools.partial(
        pl.kernel,
        mesh=mesh,
        out_type=jax.ShapeDtypeStruct((B,), jnp.float32),
        scratch_types=[
            pltpu.VMEM((NCHUNK, CHUNK), jnp.int32),
            pltpu.VMEM((NCHUNK, CHUNK), jnp.int32),
            pltpu.VMEM((NCHUNK, CHUNK), jnp.int32),
            pltpu.VMEM((NCHUNK, CHUNK), jnp.int32),
            pltpu.VMEM((ROWS_PER_PASS, TW), jnp.float32),
            pltpu.VMEM((ROWS_PER_PASS, TW), jnp.float32),
            pltpu.VMEM((BPW,), jnp.float32),
            pltpu.SemaphoreType.DMA,
        ],
        compiler_params=pltpu.CompilerParams(needs_layout_passes=False),
    )(_mf_body)
    return run(users_r, items_r, tab_u, tab_i)


def kernel(users, items, embedding_user, embedding_item):
    users_r = users.reshape(NW, NCHUNK, CHUNK)
    items_r = items.reshape(NW, NCHUNK, CHUNK)
    tab_u = _transpose_table(embedding_user.T)
    tab_i = _transpose_table(embedding_item.T)
    return _mf_call(users_r, items_r, tab_u, tab_i)


# speedup vs baseline: 1.1287x; 1.1287x over previous
import functools

import jax
import jax.numpy as jnp
from jax import lax
from jax.experimental import pallas as pl
from jax.experimental.pallas import tpu as pltpu
from jax.experimental.pallas import tpu_sc as plsc

NUM_CORES = 2
NUM_SUBCORES = 16
NW = NUM_CORES * NUM_SUBCORES
LANES = 16
B = 16384
D = 64
TW = 2 * D
V = 1000000
BPW = B // NW
CHUNK = 128
NCHUNK = BPW // CHUNK
PASS_CHUNKS = 2
ROWS_PER_PASS = PASS_CHUNKS * CHUNK
NPASS = NCHUNK // PASS_CHUNKS
BLK_PER_PASS = ROWS_PER_PASS // LANES

TRW = 512
TRG = -(-V // (2 * TRW))
PACKED_ROWS = TRG * TRW


def _tr_body(in_lo_ref, in_hi_ref, out_ref):
    eye = (lax.broadcasted_iota(jnp.int32, (D, D), 0)
           == lax.broadcasted_iota(jnp.int32, (D, D), 1)).astype(jnp.float32)
    dn = (((0,), (0,)), ((), ()))
    out_ref[:, 0:D] = lax.dot_general(
        in_lo_ref[...], eye, dn, precision=lax.Precision.HIGHEST)
    out_ref[:, D:TW] = lax.dot_general(
        in_hi_ref[...], eye, dn, precision=lax.Precision.HIGHEST)


def _transpose_table(tab_t):
    return pl.pallas_call(
        _tr_body,
        grid=(TRG,),
        in_specs=[
            pl.BlockSpec((D, TRW), lambda c: (0, 2 * c)),
            pl.BlockSpec((D, TRW), lambda c: (0, 2 * c + 1)),
        ],
        out_specs=pl.BlockSpec((TRW, TW), lambda c: (c, 0)),
        out_shape=jax.ShapeDtypeStruct((PACKED_ROWS, TW), jnp.float32),
    )(tab_t, tab_t)


def _packed_row(i):
    return lax.shift_left(lax.shift_right_logical(i, 10), 9) + (i & 511)


def _col_base(i):
    return lax.shift_left(lax.shift_right_logical(i, 9) & 1, 6)


def _mf_body(users_hbm, items_hbm, tab_u_hbm, tab_i_hbm, out_hbm,
             idx_u, idx_i, row_i, rows_u, rows_i, out_v, sem):
    wid = lax.axis_index("c") * NUM_SUBCORES + lax.axis_index("s")
    base = wid * BPW

    pltpu.sync_copy(users_hbm.at[wid], idx_u)
    pltpu.sync_copy(items_hbm.at[wid], idx_i)
    for j in range(NCHUNK):
        for k in range(CHUNK // LANES):
            s = pl.ds(k * LANES, LANES)
            row_i[j, s] = _packed_row(idx_i[j, s])

    iota = lax.iota(jnp.int32, LANES)
    grp_per_pass = PASS_CHUNKS * (CHUNK // LANES)
    for p in range(NPASS):
        item_copies = []
        for j in range(PASS_CHUNKS):
            c = p * PASS_CHUNKS + j
            item_copies.append(pltpu.async_copy(
                tab_i_hbm.at[row_i.at[c]],
                rows_i.at[pl.ds(j * CHUNK, CHUNK)], sem))

        def u_dma(v, carry, p=p):
            c = p * PASS_CHUNKS + v // (CHUNK // LANES)
            k = v % (CHUNK // LANES)
            iv = idx_u[c, pl.ds(k * LANES, LANES)]
            for l in range(LANES):
                r = v * (LANES // 2) + (l >> 1)
                pltpu.async_copy(
                    tab_u_hbm.at[iv[l]],
                    rows_u.at[r, pl.ds((l & 1) * D, D)], sem)
            return carry

        lax.fori_loop(0, grp_per_pass, u_dma, 0)
        for cp in item_copies:
            cp.wait()
        pltpu.make_async_copy(
            tab_i_hbm.at[pl.ds(0, ROWS_PER_PASS // 2)], rows_u, sem).wait()

        cb_u = lax.shift_left(iota & 1, 6)
        half_ids = lax.shift_right_logical(iota, 1)

        def dot_blk(blk, carry, p=p):
            g = p * ROWS_PER_PASS + blk * LANES
            row_ids_u = blk * (LANES // 2) + half_ids
            row_ids_i = blk * LANES + iota
            cb_i = _col_base(
                idx_i[g // CHUNK, pl.ds(g % CHUNK, LANES)])
            acc = jnp.zeros((LANES,), jnp.float32)
            for d in range(D):
                u = plsc.load_gather(rows_u, [row_ids_u, cb_u + d])
                v = plsc.load_gather(rows_i, [row_ids_i, cb_i + d])
                acc = acc + u * v
            out_v[pl.ds(g, LANES)] = 1.0 / (1.0 + jnp.exp(-acc))
            return carry

        lax.fori_loop(0, BLK_PER_PASS, dot_blk, 0)

    pltpu.sync_copy(out_v, out_hbm.at[pl.ds(base, BPW)])


@jax.jit
def _mf_call(users_r, items_r, tab_u, tab_i):
    mesh = plsc.VectorSubcoreMesh(core_axis_name="c", subcore_axis_name="s")
    run = functools.partial(
        pl.kernel,
        mesh=mesh,
        out_type=jax.ShapeDtypeStruct((B,), jnp.float32),
        scratch_types=[
            pltpu.VMEM((NCHUNK, CHUNK), jnp.int32),
            pltpu.VMEM((NCHUNK, CHUNK), jnp.int32),
            pltpu.VMEM((NCHUNK, CHUNK), jnp.int32),
            pltpu.VMEM((ROWS_PER_PASS // 2, TW), jnp.float32),
            pltpu.VMEM((ROWS_PER_PASS, TW), jnp.float32),
            pltpu.VMEM((BPW,), jnp.float32),
            pltpu.SemaphoreType.DMA,
        ],
        compiler_params=pltpu.CompilerParams(needs_layout_passes=False),
    )(_mf_body)
    return run(users_r, items_r, tab_u, tab_i)


def kernel(users, items, embedding_user, embedding_item):
    users_r = users.reshape(NW, NCHUNK, CHUNK)
    items_r = items.reshape(NW, NCHUNK, CHUNK)
    tab_i = _transpose_table(embedding_item.T)
    return _mf_call(users_r, items_r, embedding_user, tab_i)


# speedup vs baseline: 1.9731x; 1.7481x over previous
import functools

import jax
import jax.numpy as jnp
from jax import lax
from jax.experimental import pallas as pl
from jax.experimental.pallas import tpu as pltpu
from jax.experimental.pallas import tpu_sc as plsc

NUM_CORES = 2
NUM_SUBCORES = 16
NW = NUM_CORES * NUM_SUBCORES
LANES = 16
B = 16384
D = 64
TW = 2 * D
BPW = B // NW
CHUNK = 128
NCHUNK = BPW // CHUNK
PASS_CHUNKS = 2
ROWS_PER_PASS = PASS_CHUNKS * CHUNK
NPASS = NCHUNK // PASS_CHUNKS
BLK_PER_PASS = ROWS_PER_PASS // LANES
GRP_PER_PASS = ROWS_PER_PASS // LANES


def _mf_body(users_hbm, items_hbm, tab_u_hbm, tab_i_hbm, drain_hbm, out_hbm,
             idx_u, idx_i, rows_u, rows_i, out_v, sem):
    wid = lax.axis_index("c") * NUM_SUBCORES + lax.axis_index("s")
    base = wid * BPW

    pltpu.sync_copy(users_hbm.at[wid], idx_u)
    pltpu.sync_copy(items_hbm.at[wid], idx_i)

    iota = lax.iota(jnp.int32, LANES)
    kpg = CHUNK // LANES
    for p in range(NPASS):
        def row_dmas(v, carry, p=p):
            c = p * PASS_CHUNKS + v // kpg
            s = pl.ds((v % kpg) * LANES, LANES)
            iv_u = idx_u[c, s]
            iv_i = idx_i[c, s]
            for l in range(LANES):
                r = v * (LANES // 2) + (l >> 1)
                dst = pl.ds((l & 1) * D, D)
                pltpu.async_copy(tab_u_hbm.at[iv_u[l]], rows_u.at[r, dst], sem)
                pltpu.async_copy(tab_i_hbm.at[iv_i[l]], rows_i.at[r, dst], sem)
            return carry

        lax.fori_loop(0, GRP_PER_PASS, row_dmas, 0)
        pltpu.make_async_copy(drain_hbm, rows_u, sem).wait()
        pltpu.make_async_copy(drain_hbm, rows_i, sem).wait()

        cb = lax.shift_left(iota & 1, 6)
        half_ids = lax.shift_right_logical(iota, 1)

        def dot_blk(blk, carry, p=p):
            g = p * ROWS_PER_PASS + blk * LANES
            row_ids = blk * (LANES // 2) + half_ids
            acc = jnp.zeros((LANES,), jnp.float32)
            for d in range(D):
                col = cb + d
                u = plsc.load_gather(rows_u, [row_ids, col])
                v = plsc.load_gather(rows_i, [row_ids, col])
                acc = acc + u * v
            out_v[pl.ds(g, LANES)] = 1.0 / (1.0 + jnp.exp(-acc))
            return carry

        lax.fori_loop(0, BLK_PER_PASS, dot_blk, 0)

    pltpu.sync_copy(out_v, out_hbm.at[pl.ds(base, BPW)])


@jax.jit
def _mf_call(users_r, items_r, tab_u, tab_i, drain_src):
    mesh = plsc.VectorSubcoreMesh(core_axis_name="c", subcore_axis_name="s")
    run = functools.partial(
        pl.kernel,
        mesh=mesh,
        out_type=jax.ShapeDtypeStruct((B,), jnp.float32),
        scratch_types=[
            pltpu.VMEM((NCHUNK, CHUNK), jnp.int32),
            pltpu.VMEM((NCHUNK, CHUNK), jnp.int32),
            pltpu.VMEM((ROWS_PER_PASS // 2, TW), jnp.float32),
            pltpu.VMEM((ROWS_PER_PASS // 2, TW), jnp.float32),
            pltpu.VMEM((BPW,), jnp.float32),
            pltpu.SemaphoreType.DMA,
        ],
        compiler_params=pltpu.CompilerParams(needs_layout_passes=False),
    )(_mf_body)
    return run(users_r, items_r, tab_u, tab_i, drain_src)


def kernel(users, items, embedding_user, embedding_item):
    users_r = users.reshape(NW, NCHUNK, CHUNK)
    items_r = items.reshape(NW, NCHUNK, CHUNK)
    drain_src = lax.bitcast_convert_type(users, jnp.float32).reshape(
        ROWS_PER_PASS // 2, TW)
    return _mf_call(users_r, items_r, embedding_user, embedding_item,
                    drain_src)
